# Initial kernel scaffold; baseline (speedup 1.0000x reference)
import jax
import jax.numpy as jnp
from jax.experimental import pallas as pl


def kernel(z, embedding_weight):
    raise NotImplementedError("write your pallas kernel here")


# speedup vs baseline: 1.8924x; 1.8924x over previous
import functools

import jax
import jax.numpy as jnp
from jax import lax
from jax.experimental import pallas as pl
from jax.experimental.pallas import tpu as pltpu
from jax.experimental.pallas import tpu_sc as plsc

N_E = 8192
E_DIM = 256
BETA = 0.25

M = 8192
BM = 256
BN = 2048
NJ = N_E // BN


def _dist_argmin_body(z_ref, e_ref, d_ref, idx_ref, minv_ref, mini_ref):
    j = pl.program_id(1)
    zb = z_ref[...]
    eb = e_ref[pl.ds(j * BN, BN), :]

    mm = lax.dot_general(zb, eb, (((1,), (1,)), ((), ())),
                         preferred_element_type=jnp.float32)
    zsum = jnp.sum(zb * zb, axis=1, keepdims=True)
    ones = jnp.ones((1, E_DIM), jnp.float32)
    esum = lax.dot_general(ones, eb * eb, (((1,), (1,)), ((), ())),
                           preferred_element_type=jnp.float32)
    d_t = (zsum + esum) - 2.0 * mm
    d_ref[...] = d_t

    tmin = jnp.min(d_t, axis=1, keepdims=True)
    iot = lax.broadcasted_iota(jnp.int32, (BM, BN), 1)
    cand = jnp.min(jnp.where(d_t == tmin, iot, jnp.int32(2**30)),
                   axis=1, keepdims=True) + j * BN

    @pl.when(j == 0)
    def _():
        minv_ref[...] = tmin
        mini_ref[...] = cand

    @pl.when(j > 0)
    def _():
        better = tmin < minv_ref[...]
        mini_ref[...] = jnp.where(better, cand, mini_ref[...])
        minv_ref[...] = jnp.where(better, tmin, minv_ref[...])

    @pl.when(j == NJ - 1)
    def _():
        idx_ref[...] = mini_ref[...]


def _dist_argmin(z_flat, emb):
    return pl.pallas_call(
        _dist_argmin_body,
        grid=(M // BM, NJ),
        in_specs=[
            pl.BlockSpec((BM, E_DIM), lambda i, j: (i, 0)),
            pl.BlockSpec((N_E, E_DIM), lambda i, j: (0, 0)),
        ],
        out_specs=[
            pl.BlockSpec((BM, BN), lambda i, j: (i, j)),
            pl.BlockSpec((BM, 1), lambda i, j: (i, 0)),
        ],
        out_shape=[
            jax.ShapeDtypeStruct((M, N_E), jnp.float32),
            jax.ShapeDtypeStruct((M, 1), jnp.int32),
        ],
        scratch_shapes=[
            pltpu.VMEM((BM, 1), jnp.float32),
            pltpu.VMEM((BM, 1), jnp.int32),
        ],
    )(z_flat, emb)


def _sc_gather(emb, idx):
    info = plsc.get_sparse_core_info()
    nw = info.num_cores * info.num_subcores
    bw = M // nw
    mesh = plsc.VectorSubcoreMesh(core_axis_name="c", subcore_axis_name="s")

    @functools.partial(
        pl.kernel, mesh=mesh,
        out_type=jax.ShapeDtypeStruct((M, E_DIM), jnp.float32),
        scratch_types=[
            pltpu.VMEM((bw,), jnp.int32),
            pltpu.VMEM((bw, E_DIM), jnp.float32),
            pltpu.SemaphoreType.DMA,
        ],
    )
    def gather_k(table_hbm, idx_hbm, out_hbm, idx_v, rows_v, sem):
        wid = lax.axis_index("s") * info.num_cores + lax.axis_index("c")
        base = wid * bw
        pltpu.sync_copy(idx_hbm.at[pl.ds(base, bw)], idx_v)
        copies = []
        for c in range(bw // 128):
            copies.append(pltpu.async_copy(
                table_hbm.at[idx_v.at[pl.ds(c * 128, 128)]],
                rows_v.at[pl.ds(c * 128, 128)], sem))
        for cp in copies:
            cp.wait()
        pltpu.sync_copy(rows_v, out_hbm.at[pl.ds(base, bw)])

    return gather_k(emb, idx)


def _loss_st_body(zp_ref, zq_ref, o_ref, loss_ref, acc_ref):
    i = pl.program_id(0)
    nb = pl.num_programs(0)
    zp = zp_ref[...]
    zq = zq_ref[...]
    diff = zq - zp
    o_ref[...] = zp + diff
    s = jnp.sum(diff * diff)

    @pl.when(i == 0)
    def _():
        acc_ref[0] = s

    @pl.when(i > 0)
    def _():
        acc_ref[0] += s

    @pl.when(i == nb - 1)
    def _():
        m = acc_ref[0] / jnp.float32(M * E_DIM)
        loss_ref[0, 0] = m + jnp.float32(BETA) * m


def _loss_st(zp_flat, zq_flat):
    bm = 1024
    return pl.pallas_call(
        _loss_st_body,
        grid=(M // bm,),
        in_specs=[
            pl.BlockSpec((bm, E_DIM), lambda i: (i, 0)),
            pl.BlockSpec((bm, E_DIM), lambda i: (i, 0)),
        ],
        out_specs=[
            pl.BlockSpec((bm, E_DIM), lambda i: (i, 0)),
            pl.BlockSpec((1, 1), lambda i: (0, 0), memory_space=pltpu.SMEM),
        ],
        out_shape=[
            jax.ShapeDtypeStruct((M, E_DIM), jnp.float32),
            jax.ShapeDtypeStruct((1, 1), jnp.float32),
        ],
        scratch_shapes=[pltpu.SMEM((1,), jnp.float32)],
    )(zp_flat, zq_flat)


def kernel(z, embedding_weight):
    zp = jnp.transpose(z, (0, 2, 3, 1))
    z_flat = zp.reshape(-1, E_DIM)
    d, idx2 = _dist_argmin(z_flat, embedding_weight)
    idx = idx2.reshape(M)
    zq_flat = _sc_gather(embedding_weight, idx)
    out_flat, loss2 = _loss_st(z_flat, zq_flat)
    z_q = out_flat.reshape(zp.shape)
    z_q = jnp.transpose(z_q, (0, 3, 1, 2))
    return (z_q, loss2[0, 0], idx, d)


# speedup vs baseline: 1.9421x; 1.0263x over previous
import functools

import jax
import jax.numpy as jnp
from jax import lax
from jax.experimental import pallas as pl
from jax.experimental.pallas import tpu as pltpu
from jax.experimental.pallas import tpu_sc as plsc

N_E = 8192
E_DIM = 256
BETA = 0.25

M = 8192
BM = 256
BN = 2048
NJ = N_E // BN


def _dist_argmin_body(z_ref, e_ref, d_ref, idx_ref, minv_ref, mini_ref):
    j = pl.program_id(1)
    zb = z_ref[...]
    eb = e_ref[pl.ds(j * BN, BN), :]

    mm2 = lax.dot_general(zb + zb, eb, (((1,), (1,)), ((), ())),
                          preferred_element_type=jnp.float32)
    zsum = jnp.sum(zb * zb, axis=1, keepdims=True)
    ones = jnp.ones((1, E_DIM), jnp.float32)
    esum = lax.dot_general(ones, eb * eb, (((1,), (1,)), ((), ())),
                           preferred_element_type=jnp.float32)

    nc = BN // 128
    inf = jnp.full((BM, 128), jnp.inf, jnp.float32)
    zero = jnp.zeros((BM, 128), jnp.int32)
    acc_v = jnp.where(j == 0, inf, minv_ref[...])
    acc_i = jnp.where(j == 0, zero, mini_ref[...])
    for c in range(nc):
        lo, hi = c * 128, (c + 1) * 128
        v = (zsum + esum[:, lo:hi]) - mm2[:, lo:hi]
        d_ref[:, lo:hi] = v
        m = v < acc_v
        acc_i = jnp.where(m, jnp.full((BM, 128), j * nc + c, jnp.int32),
                          acc_i)
        acc_v = jnp.where(m, v, acc_v)
    minv_ref[...] = acc_v
    mini_ref[...] = acc_i

    @pl.when(j == NJ - 1)
    def _():
        rowmin = jnp.min(acc_v, axis=1, keepdims=True)
        gidx = acc_i * 128 + lax.broadcasted_iota(jnp.int32, (BM, 128), 1)
        idx_ref[...] = jnp.min(
            jnp.where(acc_v == rowmin, gidx, jnp.int32(2**30)),
            axis=1, keepdims=True)


def _dist_argmin(z_flat, emb):
    return pl.pallas_call(
        _dist_argmin_body,
        grid=(M // BM, NJ),
        in_specs=[
            pl.BlockSpec((BM, E_DIM), lambda i, j: (i, 0)),
            pl.BlockSpec((N_E, E_DIM), lambda i, j: (0, 0)),
        ],
        out_specs=[
            pl.BlockSpec((BM, BN), lambda i, j: (i, j)),
            pl.BlockSpec((BM, 1), lambda i, j: (i, 0)),
        ],
        out_shape=[
            jax.ShapeDtypeStruct((M, N_E), jnp.float32),
            jax.ShapeDtypeStruct((M, 1), jnp.int32),
        ],
        scratch_shapes=[
            pltpu.VMEM((BM, 128), jnp.float32),
            pltpu.VMEM((BM, 128), jnp.int32),
        ],
    )(z_flat, emb)


def _sc_gather(emb, idx):
    info = plsc.get_sparse_core_info()
    nw = info.num_cores * info.num_subcores
    bw = M // nw
    mesh = plsc.VectorSubcoreMesh(core_axis_name="c", subcore_axis_name="s")

    @functools.partial(
        pl.kernel, mesh=mesh,
        out_type=jax.ShapeDtypeStruct((M, E_DIM), jnp.float32),
        scratch_types=[
            pltpu.VMEM((bw,), jnp.int32),
            pltpu.VMEM((bw, E_DIM), jnp.float32),
            pltpu.SemaphoreType.DMA,
        ],
    )
    def gather_k(table_hbm, idx_hbm, out_hbm, idx_v, rows_v, sem):
        wid = lax.axis_index("s") * info.num_cores + lax.axis_index("c")
        base = wid * bw
        pltpu.sync_copy(idx_hbm.at[pl.ds(base, bw)], idx_v)
        copies = []
        for c in range(bw // 128):
            copies.append(pltpu.async_copy(
                table_hbm.at[idx_v.at[pl.ds(c * 128, 128)]],
                rows_v.at[pl.ds(c * 128, 128)], sem))
        for cp in copies:
            cp.wait()
        pltpu.sync_copy(rows_v, out_hbm.at[pl.ds(base, bw)])

    return gather_k(emb, idx)


def _loss_st_body(zp_ref, zq_ref, o_ref, loss_ref, acc_ref):
    i = pl.program_id(0)
    nb = pl.num_programs(0)
    zp = zp_ref[...]
    zq = zq_ref[...]
    diff = zq - zp
    o_ref[...] = zp + diff
    s = jnp.sum(diff * diff)

    @pl.when(i == 0)
    def _():
        acc_ref[0] = s

    @pl.when(i > 0)
    def _():
        acc_ref[0] += s

    @pl.when(i == nb - 1)
    def _():
        m = acc_ref[0] / jnp.float32(M * E_DIM)
        loss_ref[0, 0] = m + jnp.float32(BETA) * m


def _loss_st(zp_flat, zq_flat):
    bm = 1024
    return pl.pallas_call(
        _loss_st_body,
        grid=(M // bm,),
        in_specs=[
            pl.BlockSpec((bm, E_DIM), lambda i: (i, 0)),
            pl.BlockSpec((bm, E_DIM), lambda i: (i, 0)),
        ],
        out_specs=[
            pl.BlockSpec((bm, E_DIM), lambda i: (i, 0)),
            pl.BlockSpec((1, 1), lambda i: (0, 0), memory_space=pltpu.SMEM),
        ],
        out_shape=[
            jax.ShapeDtypeStruct((M, E_DIM), jnp.float32),
            jax.ShapeDtypeStruct((1, 1), jnp.float32),
        ],
        scratch_shapes=[pltpu.SMEM((1,), jnp.float32)],
    )(zp_flat, zq_flat)


def kernel(z, embedding_weight):
    zp = jnp.transpose(z, (0, 2, 3, 1))
    z_flat = zp.reshape(-1, E_DIM)
    d, idx2 = _dist_argmin(z_flat, embedding_weight)
    idx = idx2.reshape(M)
    zq_flat = _sc_gather(embedding_weight, idx)
    out_flat, loss2 = _loss_st(z_flat, zq_flat)
    z_q = out_flat.reshape(zp.shape)
    z_q = jnp.transpose(z_q, (0, 3, 1, 2))
    return (z_q, loss2[0, 0], idx, d)
